# baseline (speedup 1.0000x reference)
import functools

import jax
import jax.numpy as jnp
from jax import lax
from jax.experimental import pallas as pl
from jax.experimental.pallas import tpu as pltpu
from jax.experimental.pallas import tpu_sc as plsc

N, L, D = 64, 1024, 768
NUM_BINS = 10
LEN_KEEP = 256


def _fused_body(x_ref, tri_ref, idr_ref, mask_ref, gidx_ref):
    xb = x_ref[0]
    mn = jnp.min(xb, axis=-1, keepdims=True)
    mx = jnp.max(xb, axis=-1, keepdims=True)
    norm = (xb - mn) / (mx - mn + 1e-19)
    z = norm * (NUM_BINS - 1)
    cle = []
    for b in range(NUM_BINS - 1):
        cle.append(jnp.sum((z < float(b + 1)).astype(jnp.float32), axis=-1,
                           keepdims=True))
    t = []
    prev = jnp.zeros((L, 1), jnp.float32)
    for b in range(NUM_BINS):
        cur = cle[b] if b < NUM_BINS - 1 else jnp.full((L, 1), float(D))
        c = cur - prev
        prev = cur
        p = c / float(D)
        t.append(p * jnp.log(p + 1e-9))
    u0 = (t[0] + t[8]) + t[4]
    u1 = (t[1] + t[9]) + t[5]
    u2 = t[2] + t[6]
    u3 = t[3] + t[7]
    ecol = -((u0 + u2) + (u1 + u3))

    erow = jnp.transpose(ecol)
    before = jnp.where(erow > ecol, 1.0,
                       jnp.where(erow == ecol, tri_ref[...], 0.0))
    rank = jnp.sum(before, axis=1, keepdims=True)
    rank_row = jnp.transpose(rank)
    idr_ref[0] = rank_row.astype(jnp.int32)
    mask_ref[0] = (rank_row >= float(LEN_KEEP)).astype(jnp.float32)
    j = lax.broadcasted_iota(jnp.int32, (1, LEN_KEEP), 1).astype(jnp.float32)
    onehot = (rank == j).astype(jnp.float32)
    lcol = lax.broadcasted_iota(jnp.int32, (L, 1), 0).astype(jnp.float32)
    ids = jnp.sum(onehot * lcol, axis=0, keepdims=True)
    n = pl.program_id(0)
    gidx_ref[0] = ids.astype(jnp.int32) + n * L


def _rank_stage(x):
    tri = (jnp.arange(L)[None, :] < jnp.arange(L)[:, None]).astype(jnp.float32)
    return pl.pallas_call(
        _fused_body,
        grid=(N,),
        in_specs=[
            pl.BlockSpec((1, L, D), lambda n: (n, 0, 0)),
            pl.BlockSpec((L, L), lambda n: (0, 0)),
        ],
        out_specs=[
            pl.BlockSpec((1, 1, L), lambda n: (n, 0, 0)),
            pl.BlockSpec((1, 1, L), lambda n: (n, 0, 0)),
            pl.BlockSpec((1, 1, LEN_KEEP), lambda n: (n, 0, 0)),
        ],
        out_shape=[
            jax.ShapeDtypeStruct((N, 1, L), jnp.int32),
            jax.ShapeDtypeStruct((N, 1, L), jnp.float32),
            jax.ShapeDtypeStruct((N, 1, LEN_KEEP), jnp.int32),
        ],
    )(x, tri)


_ROWS = N * LEN_KEEP
_CH = 64


@functools.cache
def _make_sc_gather():
    info = plsc.get_sparse_core_info()
    nc, ns = info.num_cores, info.num_subcores
    rpw = _ROWS // (nc * ns)
    nchunk = rpw // _CH

    @functools.partial(
        pl.kernel,
        out_type=jax.ShapeDtypeStruct((_ROWS, D), jnp.float32),
        mesh=plsc.VectorSubcoreMesh(core_axis_name="c", subcore_axis_name="s"),
        scratch_types=[
            pltpu.VMEM((rpw,), jnp.int32),
            pltpu.VMEM((2, _CH, D), jnp.float32),
            pltpu.SemaphoreType.DMA,
            pltpu.SemaphoreType.DMA,
            pltpu.SemaphoreType.DMA,
            pltpu.SemaphoreType.DMA,
        ],
    )
    def _sc_gather(x_hbm, gidx_hbm, out_hbm, idx_v, rows_v,
                   gsem0, gsem1, osem0, osem1):
        wid = lax.axis_index("s") * nc + lax.axis_index("c")
        base = wid * rpw
        gsem = (gsem0, gsem1)
        osem = (osem0, osem1)
        pltpu.sync_copy(gidx_hbm.at[pl.ds(base, rpw)], idx_v)
        gathers = [None, None]
        writes = [None, None]
        for c in range(nchunk):
            buf = c % 2
            if writes[buf] is not None:
                writes[buf].wait()
            gathers[buf] = pltpu.async_copy(
                x_hbm.at[idx_v.at[pl.ds(c * _CH, _CH)]], rows_v.at[buf],
                gsem[buf])
            if c >= 1:
                pb = 1 - buf
                gathers[pb].wait()
                writes[pb] = pltpu.async_copy(
                    rows_v.at[pb],
                    out_hbm.at[pl.ds(base + (c - 1) * _CH, _CH)], osem[pb])
        lb = (nchunk - 1) % 2
        gathers[lb].wait()
        writes[lb] = pltpu.async_copy(
            rows_v.at[lb],
            out_hbm.at[pl.ds(base + (nchunk - 1) * _CH, _CH)], osem[lb])
        writes[1 - lb].wait()
        writes[lb].wait()

    return _sc_gather


@jax.jit
def kernel(x):
    idr, mask, gidx = _rank_stage(x)
    rows = _make_sc_gather()(x.reshape(N * L, D), gidx.reshape(_ROWS))
    x_masked = rows.reshape(N, LEN_KEEP, D)
    return (x_masked, mask.reshape(N, L), idr.reshape(N, L))


# speedup vs baseline: 1.0463x; 1.0463x over previous
import functools

import jax
import jax.numpy as jnp
from jax import lax
from jax.experimental import pallas as pl
from jax.experimental.pallas import tpu as pltpu
from jax.experimental.pallas import tpu_sc as plsc

N, L, D = 64, 1024, 768
NUM_BINS = 10
LEN_KEEP = 256


def _fused_body(x_ref, idr_ref, mask_ref, gidx_ref):
    for s in range(_SPB):
        _one_sample(x_ref, idr_ref, mask_ref, gidx_ref, s)


_SPB = 2


def _one_sample(x_ref, idr_ref, mask_ref, gidx_ref, s):
    xb = x_ref[s]
    mn = jnp.min(xb, axis=-1, keepdims=True)
    mx = jnp.max(xb, axis=-1, keepdims=True)
    norm = (xb - mn) / (mx - mn + 1e-19)
    z = norm * (NUM_BINS - 1)
    cle = []
    for b in range(NUM_BINS - 1):
        cle.append(jnp.sum((z < float(b + 1)).astype(jnp.float32), axis=-1,
                           keepdims=True))
    t = []
    prev = jnp.zeros((L, 1), jnp.float32)
    for b in range(NUM_BINS):
        cur = cle[b] if b < NUM_BINS - 1 else jnp.full((L, 1), float(D))
        c = cur - prev
        prev = cur
        p = c / float(D)
        t.append(p * jnp.log(p + 1e-9))
    u0 = (t[0] + t[8]) + t[4]
    u1 = (t[1] + t[9]) + t[5]
    u2 = t[2] + t[6]
    u3 = t[3] + t[7]
    ecol = -((u0 + u2) + (u1 + u3))

    erow = jnp.transpose(ecol)
    lidx = lax.broadcasted_iota(jnp.int32, (L, L), 0)
    midx = lax.broadcasted_iota(jnp.int32, (L, L), 1)
    before = (erow > ecol) | ((erow == ecol) & (midx < lidx))
    rank = jnp.sum(before.astype(jnp.float32), axis=1, keepdims=True)
    rank_row = jnp.transpose(rank)
    idr_ref[s] = rank_row.astype(jnp.int32)
    mask_ref[s] = (rank_row >= float(LEN_KEEP)).astype(jnp.float32)
    j = lax.broadcasted_iota(jnp.int32, (1, LEN_KEEP), 1).astype(jnp.float32)
    onehot = (rank == j).astype(jnp.float32)
    lcol = lax.broadcasted_iota(jnp.int32, (L, 1), 0).astype(jnp.float32)
    ids = jnp.sum(onehot * lcol, axis=0, keepdims=True)
    n = pl.program_id(0) * _SPB + s
    gidx_ref[s] = ids.astype(jnp.int32) + n * L


def _rank_stage(x):
    return pl.pallas_call(
        _fused_body,
        grid=(N // _SPB,),
        in_specs=[pl.BlockSpec((_SPB, L, D), lambda n: (n, 0, 0))],
        out_specs=[
            pl.BlockSpec((_SPB, 1, L), lambda n: (n, 0, 0)),
            pl.BlockSpec((_SPB, 1, L), lambda n: (n, 0, 0)),
            pl.BlockSpec((_SPB, 1, LEN_KEEP), lambda n: (n, 0, 0)),
        ],
        out_shape=[
            jax.ShapeDtypeStruct((N, 1, L), jnp.int32),
            jax.ShapeDtypeStruct((N, 1, L), jnp.float32),
            jax.ShapeDtypeStruct((N, 1, LEN_KEEP), jnp.int32),
        ],
    )(x)


_ROWS = N * LEN_KEEP
_CH = 64


@functools.cache
def _make_sc_gather():
    info = plsc.get_sparse_core_info()
    nc, ns = info.num_cores, info.num_subcores
    rpw = _ROWS // (nc * ns)
    nchunk = rpw // _CH

    @functools.partial(
        pl.kernel,
        out_type=jax.ShapeDtypeStruct((_ROWS, D), jnp.float32),
        mesh=plsc.VectorSubcoreMesh(core_axis_name="c", subcore_axis_name="s"),
        scratch_types=[
            pltpu.VMEM((rpw,), jnp.int32),
            pltpu.VMEM((2, _CH, D), jnp.float32),
            pltpu.SemaphoreType.DMA,
            pltpu.SemaphoreType.DMA,
            pltpu.SemaphoreType.DMA,
            pltpu.SemaphoreType.DMA,
        ],
    )
    def _sc_gather(x_hbm, gidx_hbm, out_hbm, idx_v, rows_v,
                   gsem0, gsem1, osem0, osem1):
        wid = lax.axis_index("s") * nc + lax.axis_index("c")
        base = wid * rpw
        gsem = (gsem0, gsem1)
        osem = (osem0, osem1)
        pltpu.sync_copy(gidx_hbm.at[pl.ds(base, rpw)], idx_v)
        gathers = [None, None]
        writes = [None, None]
        for c in range(nchunk):
            buf = c % 2
            if writes[buf] is not None:
                writes[buf].wait()
            gathers[buf] = pltpu.async_copy(
                x_hbm.at[idx_v.at[pl.ds(c * _CH, _CH)]], rows_v.at[buf],
                gsem[buf])
            if c >= 1:
                pb = 1 - buf
                gathers[pb].wait()
                writes[pb] = pltpu.async_copy(
                    rows_v.at[pb],
                    out_hbm.at[pl.ds(base + (c - 1) * _CH, _CH)], osem[pb])
        lb = (nchunk - 1) % 2
        gathers[lb].wait()
        writes[lb] = pltpu.async_copy(
            rows_v.at[lb],
            out_hbm.at[pl.ds(base + (nchunk - 1) * _CH, _CH)], osem[lb])
        writes[1 - lb].wait()
        writes[lb].wait()

    return _sc_gather


@jax.jit
def kernel(x):
    idr, mask, gidx = _rank_stage(x)
    rows = _make_sc_gather()(x.reshape(N * L, D), gidx.reshape(_ROWS))
    x_masked = rows.reshape(N, LEN_KEEP, D)
    return (x_masked, mask.reshape(N, L), idr.reshape(N, L))


# speedup vs baseline: 1.0509x; 1.0044x over previous
import functools

import jax
import jax.numpy as jnp
from jax import lax
from jax.experimental import pallas as pl
from jax.experimental.pallas import tpu as pltpu
from jax.experimental.pallas import tpu_sc as plsc

N, L, D = 64, 1024, 768
NUM_BINS = 10
LEN_KEEP = 256


def _fused_body(x_ref, idr_ref, mask_ref, gidx_ref):
    for s in range(_SPB):
        _one_sample(x_ref, idr_ref, mask_ref, gidx_ref, s)


_SPB = 4


def _one_sample(x_ref, idr_ref, mask_ref, gidx_ref, s):
    xb = x_ref[s]
    mn = jnp.min(xb, axis=-1, keepdims=True)
    mx = jnp.max(xb, axis=-1, keepdims=True)
    norm = (xb - mn) / (mx - mn + 1e-19)
    z = norm * (NUM_BINS - 1)
    cle = []
    for b in range(NUM_BINS - 1):
        cle.append(jnp.sum((z < float(b + 1)).astype(jnp.float32), axis=-1,
                           keepdims=True))
    t = []
    prev = jnp.zeros((L, 1), jnp.float32)
    for b in range(NUM_BINS):
        cur = cle[b] if b < NUM_BINS - 1 else jnp.full((L, 1), float(D))
        c = cur - prev
        prev = cur
        p = c / float(D)
        t.append(p * jnp.log(p + 1e-9))
    u0 = (t[0] + t[8]) + t[4]
    u1 = (t[1] + t[9]) + t[5]
    u2 = t[2] + t[6]
    u3 = t[3] + t[7]
    ecol = -((u0 + u2) + (u1 + u3))

    erow = jnp.transpose(ecol)
    lidx = lax.broadcasted_iota(jnp.int32, (L, L), 0)
    midx = lax.broadcasted_iota(jnp.int32, (L, L), 1)
    before = (erow > ecol) | ((erow == ecol) & (midx < lidx))
    rank = jnp.sum(before.astype(jnp.float32), axis=1, keepdims=True)
    rank_row = jnp.transpose(rank)
    idr_ref[s] = rank_row.astype(jnp.int32)
    mask_ref[s] = (rank_row >= float(LEN_KEEP)).astype(jnp.float32)
    j = lax.broadcasted_iota(jnp.int32, (1, LEN_KEEP), 1).astype(jnp.float32)
    onehot = (rank == j).astype(jnp.float32)
    lcol = lax.broadcasted_iota(jnp.int32, (L, 1), 0).astype(jnp.float32)
    ids = jnp.sum(onehot * lcol, axis=0, keepdims=True)
    n = pl.program_id(0) * _SPB + s
    gidx_ref[s] = ids.astype(jnp.int32) + n * L


def _rank_stage(x):
    return pl.pallas_call(
        _fused_body,
        grid=(N // _SPB,),
        in_specs=[pl.BlockSpec((_SPB, L, D), lambda n: (n, 0, 0))],
        out_specs=[
            pl.BlockSpec((_SPB, 1, L), lambda n: (n, 0, 0)),
            pl.BlockSpec((_SPB, 1, L), lambda n: (n, 0, 0)),
            pl.BlockSpec((_SPB, 1, LEN_KEEP), lambda n: (n, 0, 0)),
        ],
        out_shape=[
            jax.ShapeDtypeStruct((N, 1, L), jnp.int32),
            jax.ShapeDtypeStruct((N, 1, L), jnp.float32),
            jax.ShapeDtypeStruct((N, 1, LEN_KEEP), jnp.int32),
        ],
    )(x)


_ROWS = N * LEN_KEEP
_CH = 64


@functools.cache
def _make_sc_gather():
    info = plsc.get_sparse_core_info()
    nc, ns = info.num_cores, info.num_subcores
    rpw = _ROWS // (nc * ns)
    nchunk = rpw // _CH

    @functools.partial(
        pl.kernel,
        out_type=jax.ShapeDtypeStruct((_ROWS, D), jnp.float32),
        mesh=plsc.VectorSubcoreMesh(core_axis_name="c", subcore_axis_name="s"),
        scratch_types=[
            pltpu.VMEM((rpw,), jnp.int32),
            pltpu.VMEM((2, _CH, D), jnp.float32),
            pltpu.SemaphoreType.DMA,
            pltpu.SemaphoreType.DMA,
            pltpu.SemaphoreType.DMA,
            pltpu.SemaphoreType.DMA,
        ],
    )
    def _sc_gather(x_hbm, gidx_hbm, out_hbm, idx_v, rows_v,
                   gsem0, gsem1, osem0, osem1):
        wid = lax.axis_index("s") * nc + lax.axis_index("c")
        base = wid * rpw
        gsem = (gsem0, gsem1)
        osem = (osem0, osem1)
        pltpu.sync_copy(gidx_hbm.at[pl.ds(base, rpw)], idx_v)
        gathers = [None, None]
        writes = [None, None]
        for c in range(nchunk):
            buf = c % 2
            if writes[buf] is not None:
                writes[buf].wait()
            gathers[buf] = pltpu.async_copy(
                x_hbm.at[idx_v.at[pl.ds(c * _CH, _CH)]], rows_v.at[buf],
                gsem[buf])
            if c >= 1:
                pb = 1 - buf
                gathers[pb].wait()
                writes[pb] = pltpu.async_copy(
                    rows_v.at[pb],
                    out_hbm.at[pl.ds(base + (c - 1) * _CH, _CH)], osem[pb])
        lb = (nchunk - 1) % 2
        gathers[lb].wait()
        writes[lb] = pltpu.async_copy(
            rows_v.at[lb],
            out_hbm.at[pl.ds(base + (nchunk - 1) * _CH, _CH)], osem[lb])
        writes[1 - lb].wait()
        writes[lb].wait()

    return _sc_gather


@jax.jit
def kernel(x):
    idr, mask, gidx = _rank_stage(x)
    rows = _make_sc_gather()(x.reshape(N * L, D), gidx.reshape(_ROWS))
    x_masked = rows.reshape(N, LEN_KEEP, D)
    return (x_masked, mask.reshape(N, L), idr.reshape(N, L))
